# Initial kernel scaffold; baseline (speedup 1.0000x reference)
import jax
import jax.numpy as jnp
from jax.experimental import pallas as pl


def kernel(x, edge_index, W1, b1, W2, b2, W3, b3):
    raise NotImplementedError("write your pallas kernel here")


# speedup vs baseline: 3.3099x; 3.3099x over previous
import functools

import jax
import jax.numpy as jnp
from jax import lax
from jax.experimental import pallas as pl
from jax.experimental.pallas import tpu as pltpu
from jax.experimental.pallas import tpu_sc as plsc

N = 10000
E = 320000
NSC = 2
NTILE = 16
NW = NSC * NTILE
CHUNK = 128
EPW = E // NW
EPW_P = 10240
NCHUNK = EPW_P // CHUNK
NP = 10240
ROWS_PT = NP // NTILE
ZCH = ROWS_PT // CHUNK

TM = 1000


def _mm_body(x_ref, w_ref, o_ref):
    o_ref[...] = jnp.dot(x_ref[...], w_ref[...],
                         preferred_element_type=jnp.float32)


def _fused_mm_body(acc_ref, b_ref, w_ref, o_ref):
    s = acc_ref[0] + acc_ref[1] + b_ref[...]
    h = jnp.where(s >= 0, s, 0.25 * s)
    o_ref[...] = jnp.dot(h, w_ref[...], preferred_element_type=jnp.float32)


def _fin_body(acc_ref, b_ref, o_ref):
    s = acc_ref[0] + acc_ref[1] + b_ref[...]
    o_ref[...] = jnp.where(s >= 0, s, 0.25 * s)


def _mm(x, w):
    din, dout = w.shape
    return pl.pallas_call(
        _mm_body,
        grid=(N // TM,),
        in_specs=[
            pl.BlockSpec((TM, din), lambda i: (i, 0)),
            pl.BlockSpec((din, dout), lambda i: (0, 0)),
        ],
        out_specs=pl.BlockSpec((TM, dout), lambda i: (i, 0)),
        out_shape=jax.ShapeDtypeStruct((N, dout), jnp.float32),
    )(x, w)


def _fused_mm(acc, b, w):
    din, dout = w.shape
    return pl.pallas_call(
        _fused_mm_body,
        grid=(N // TM,),
        in_specs=[
            pl.BlockSpec((2, TM, din), lambda i: (0, i, 0)),
            pl.BlockSpec((1, din), lambda i: (0, 0)),
            pl.BlockSpec((din, dout), lambda i: (0, 0)),
        ],
        out_specs=pl.BlockSpec((TM, dout), lambda i: (i, 0)),
        out_shape=jax.ShapeDtypeStruct((N, dout), jnp.float32),
    )(acc[:, :N, :], b.reshape(1, din), w)


def _fin(acc, b):
    d = acc.shape[-1]
    return pl.pallas_call(
        _fin_body,
        grid=(N // TM,),
        in_specs=[
            pl.BlockSpec((2, TM, d), lambda i: (0, i, 0)),
            pl.BlockSpec((1, d), lambda i: (0, 0)),
        ],
        out_specs=pl.BlockSpec((TM, d), lambda i: (i, 0)),
        out_shape=jax.ShapeDtypeStruct((N, d), jnp.float32),
    )(acc[:, :N, :], b.reshape(1, d))


@functools.lru_cache(maxsize=None)
def _make_prop(d):
    mesh = plsc.VectorSubcoreMesh(core_axis_name="c", subcore_axis_name="s")

    @functools.partial(
        pl.kernel,
        mesh=mesh,
        out_type=jax.ShapeDtypeStruct((NSC, NP, d), jnp.float32),
        scratch_types=[
            pltpu.VMEM((NCHUNK, CHUNK), jnp.int32),
            pltpu.VMEM((NCHUNK, CHUNK), jnp.int32),
            pltpu.VMEM((CHUNK, d), jnp.float32),
            pltpu.VMEM_SHARED((NP, d), jnp.float32),
            pltpu.SemaphoreType.DMA,
        ],
    )
    def prop(t_hbm, src_hbm, dst_hbm, zero_hbm, out_hbm,
             src_v, dst_v, rows_v, acc_sh, sem):
        c = lax.axis_index("c")
        s = lax.axis_index("s")
        wid = c * NTILE + s
        r0 = s * ROWS_PT

        pltpu.sync_copy(zero_hbm, rows_v)
        for z in range(ZCH):
            pltpu.sync_copy(rows_v, acc_sh.at[pl.ds(r0 + z * CHUNK, CHUNK)])

        pltpu.sync_copy(src_hbm.at[wid], src_v)
        pltpu.sync_copy(dst_hbm.at[wid], dst_v)
        plsc.subcore_barrier()

        def body(j, carry):
            pltpu.async_copy(t_hbm.at[src_v.at[j]], rows_v, sem).wait()
            pltpu.sync_copy(rows_v, acc_sh.at[dst_v.at[j]], add=True)
            return carry

        lax.fori_loop(0, NCHUNK, body, 0)
        plsc.subcore_barrier()

        for z in range(ZCH):
            rz = r0 + z * CHUNK
            pltpu.sync_copy(acc_sh.at[pl.ds(rz, CHUNK)], rows_v)
            pltpu.sync_copy(rows_v, out_hbm.at[c, pl.ds(rz, CHUNK)])

    return prop


def _pad_edges(idx, fill):
    w = idx.reshape(NW, EPW)
    pad = jnp.full((NW, EPW_P - EPW), fill, jnp.int32)
    return jnp.concatenate([w, pad], axis=1).reshape(NW, NCHUNK, CHUNK)


def kernel(x, edge_index, W1, b1, W2, b2, W3, b3):
    src = _pad_edges(edge_index[0].astype(jnp.int32), 0)
    dst = _pad_edges(edge_index[1].astype(jnp.int32), N)
    z128 = jnp.zeros((CHUNK, 128), jnp.float32)
    W3p = jnp.concatenate([W3, jnp.zeros((128, 64), jnp.float32)], axis=1)

    prop128 = _make_prop(128)

    t1 = _mm(x, W1)
    s1 = prop128(t1, src, dst, z128)
    t2 = _fused_mm(s1, b1, W2)
    s2 = prop128(t2, src, dst, z128)
    t3 = _fused_mm(s2, b2, W3p)
    s3 = prop128(t3, src, dst, z128)
    return _fin(s3[:, :, :64], b3)


# speedup vs baseline: 3.6305x; 1.0969x over previous
import functools

import jax
import jax.numpy as jnp
from jax import lax
from jax.experimental import pallas as pl
from jax.experimental.pallas import tpu as pltpu
from jax.experimental.pallas import tpu_sc as plsc

N = 10000
E = 320000
NSC = 2
NTILE = 16
NW = NSC * NTILE
CHUNK = 128
EPW = E // NW
EPW_P = 10240
NCHUNK = EPW_P // CHUNK
GCHUNK = 8
NGRP = NCHUNK // GCHUNK
NP = 10240
ROWS_PT = NP // NTILE
ZCH = ROWS_PT // CHUNK
NBUF = 2

TM = 1000


def _mm_body(x_ref, w_ref, o_ref):
    o_ref[...] = jnp.dot(x_ref[...], w_ref[...],
                         preferred_element_type=jnp.float32)


def _fused_mm_body(acc_ref, b_ref, w_ref, o_ref):
    s = acc_ref[0] + acc_ref[1] + b_ref[...]
    h = jnp.where(s >= 0, s, 0.25 * s)
    o_ref[...] = jnp.dot(h, w_ref[...], preferred_element_type=jnp.float32)


def _fin_body(acc_ref, b_ref, o_ref):
    s = acc_ref[0] + acc_ref[1] + b_ref[...]
    o_ref[...] = jnp.where(s >= 0, s, 0.25 * s)


def _mm(x, w):
    din, dout = w.shape
    return pl.pallas_call(
        _mm_body,
        grid=(N // TM,),
        in_specs=[
            pl.BlockSpec((TM, din), lambda i: (i, 0)),
            pl.BlockSpec((din, dout), lambda i: (0, 0)),
        ],
        out_specs=pl.BlockSpec((TM, dout), lambda i: (i, 0)),
        out_shape=jax.ShapeDtypeStruct((N, dout), jnp.float32),
    )(x, w)


def _fused_mm(acc, b, w):
    din, dout = w.shape
    return pl.pallas_call(
        _fused_mm_body,
        grid=(N // TM,),
        in_specs=[
            pl.BlockSpec((2, TM, din), lambda i: (0, i, 0)),
            pl.BlockSpec((1, din), lambda i: (0, 0)),
            pl.BlockSpec((din, dout), lambda i: (0, 0)),
        ],
        out_specs=pl.BlockSpec((TM, dout), lambda i: (i, 0)),
        out_shape=jax.ShapeDtypeStruct((N, dout), jnp.float32),
    )(acc[:, :N, :], b.reshape(1, din), w)


def _fin(acc, b):
    d = acc.shape[-1]
    return pl.pallas_call(
        _fin_body,
        grid=(N // TM,),
        in_specs=[
            pl.BlockSpec((2, TM, d), lambda i: (0, i, 0)),
            pl.BlockSpec((1, d), lambda i: (0, 0)),
        ],
        out_specs=pl.BlockSpec((TM, d), lambda i: (i, 0)),
        out_shape=jax.ShapeDtypeStruct((N, d), jnp.float32),
    )(acc[:, :N, :], b.reshape(1, d))


@functools.lru_cache(maxsize=None)
def _make_prop(d):
    mesh = plsc.VectorSubcoreMesh(core_axis_name="c", subcore_axis_name="s")

    @functools.partial(
        pl.kernel,
        mesh=mesh,
        out_type=jax.ShapeDtypeStruct((NSC, NP, d), jnp.float32),
        scratch_types=[
            pltpu.VMEM((GCHUNK, CHUNK), jnp.int32),
            pltpu.VMEM((GCHUNK, CHUNK), jnp.int32),
            pltpu.VMEM((NBUF, CHUNK, d), jnp.float32),
            pltpu.VMEM_SHARED((NP, d), jnp.float32),
            pltpu.SemaphoreType.DMA((NBUF,)),
        ],
    )
    def prop(t_hbm, src_hbm, dst_hbm, zero_hbm, out_hbm,
             src_v, dst_v, rows_v, acc_sh, sem):
        c = lax.axis_index("c")
        s = lax.axis_index("s")
        wid = c * NTILE + s
        r0 = s * ROWS_PT

        pltpu.sync_copy(zero_hbm, rows_v.at[0])
        for z in range(ZCH):
            pltpu.sync_copy(rows_v.at[0], acc_sh.at[pl.ds(r0 + z * CHUNK, CHUNK)])
        plsc.subcore_barrier()

        @pl.loop(0, NGRP)
        def _(g):
            g8 = pl.multiple_of(g * GCHUNK, GCHUNK)
            pltpu.sync_copy(src_hbm.at[wid, pl.ds(g8, GCHUNK)], src_v)
            pltpu.sync_copy(dst_hbm.at[wid, pl.ds(g8, GCHUNK)], dst_v)
            pltpu.async_copy(t_hbm.at[src_v.at[0]], rows_v.at[0], sem.at[0])
            for jj in range(GCHUNK):
                b = jj % NBUF
                if jj + 1 < GCHUNK:
                    nb = (jj + 1) % NBUF
                    pltpu.async_copy(
                        t_hbm.at[src_v.at[jj + 1]], rows_v.at[nb], sem.at[nb])
                pltpu.make_async_copy(
                    t_hbm.at[src_v.at[jj]], rows_v.at[b], sem.at[b]).wait()
                pltpu.sync_copy(rows_v.at[b], acc_sh.at[dst_v.at[jj]], add=True)

        plsc.subcore_barrier()

        for z in range(ZCH):
            rz = r0 + z * CHUNK
            pltpu.sync_copy(acc_sh.at[pl.ds(rz, CHUNK)], rows_v.at[0])
            pltpu.sync_copy(rows_v.at[0], out_hbm.at[c, pl.ds(rz, CHUNK)])

    return prop


def _pad_edges(idx, fill):
    w = idx.reshape(NW, EPW)
    pad = jnp.full((NW, EPW_P - EPW), fill, jnp.int32)
    return jnp.concatenate([w, pad], axis=1).reshape(NW, NCHUNK, CHUNK)


def kernel(x, edge_index, W1, b1, W2, b2, W3, b3):
    src = _pad_edges(edge_index[0].astype(jnp.int32), 0)
    dst = _pad_edges(edge_index[1].astype(jnp.int32), N)
    z128 = jnp.zeros((CHUNK, 128), jnp.float32)
    W3p = jnp.concatenate([W3, jnp.zeros((128, 64), jnp.float32)], axis=1)

    prop128 = _make_prop(128)

    t1 = _mm(x, W1)
    s1 = prop128(t1, src, dst, z128)
    t2 = _fused_mm(s1, b1, W2)
    s2 = prop128(t2, src, dst, z128)
    t3 = _fused_mm(s2, b2, W3p)
    s3 = prop128(t3, src, dst, z128)
    return _fin(s3[:, :, :64], b3)


# speedup vs baseline: 3.6335x; 1.0008x over previous
import functools

import jax
import jax.numpy as jnp
from jax import lax
from jax.experimental import pallas as pl
from jax.experimental.pallas import tpu as pltpu
from jax.experimental.pallas import tpu_sc as plsc

N = 10000
E = 320000
NSC = 2
NTILE = 16
NW = NSC * NTILE
CHUNK = 128
EPW = E // NW
EPW_P = 10240
NCHUNK = EPW_P // CHUNK
GCHUNK = 8
NGRP = NCHUNK // GCHUNK
NP = 10240
ROWS_PT = NP // NTILE
ZCH = ROWS_PT // CHUNK
NBUF = 2

TM = 1000


def _mm_body(x_ref, w_ref, o_ref):
    o_ref[...] = jnp.dot(x_ref[...], w_ref[...],
                         preferred_element_type=jnp.float32)


def _fused_mm_body(acc_ref, b_ref, w_ref, o_ref):
    s = acc_ref[0] + acc_ref[1] + b_ref[...]
    h = jnp.where(s >= 0, s, 0.25 * s)
    o_ref[...] = jnp.dot(h, w_ref[...], preferred_element_type=jnp.float32)


def _fin_body(acc_ref, b_ref, o_ref):
    s = acc_ref[0] + acc_ref[1] + b_ref[...]
    o_ref[...] = jnp.where(s >= 0, s, 0.25 * s)


def _mm(x, w):
    din, dout = w.shape
    return pl.pallas_call(
        _mm_body,
        grid=(N // TM,),
        in_specs=[
            pl.BlockSpec((TM, din), lambda i: (i, 0)),
            pl.BlockSpec((din, dout), lambda i: (0, 0)),
        ],
        out_specs=pl.BlockSpec((TM, dout), lambda i: (i, 0)),
        out_shape=jax.ShapeDtypeStruct((N, dout), jnp.float32),
    )(x, w)


def _fused_mm(acc, b, w):
    din, dout = w.shape
    return pl.pallas_call(
        _fused_mm_body,
        grid=(N // TM,),
        in_specs=[
            pl.BlockSpec((2, TM, din), lambda i: (0, i, 0)),
            pl.BlockSpec((1, din), lambda i: (0, 0)),
            pl.BlockSpec((din, dout), lambda i: (0, 0)),
        ],
        out_specs=pl.BlockSpec((TM, dout), lambda i: (i, 0)),
        out_shape=jax.ShapeDtypeStruct((N, dout), jnp.float32),
    )(acc[:, :N, :], b.reshape(1, din), w)


def _fin(acc, b):
    d = acc.shape[-1]
    return pl.pallas_call(
        _fin_body,
        grid=(N // TM,),
        in_specs=[
            pl.BlockSpec((2, TM, d), lambda i: (0, i, 0)),
            pl.BlockSpec((1, d), lambda i: (0, 0)),
        ],
        out_specs=pl.BlockSpec((TM, d), lambda i: (i, 0)),
        out_shape=jax.ShapeDtypeStruct((N, d), jnp.float32),
    )(acc[:, :N, :], b.reshape(1, d))


@functools.lru_cache(maxsize=None)
def _make_prop(d):
    mesh = plsc.VectorSubcoreMesh(core_axis_name="c", subcore_axis_name="s")

    @functools.partial(
        pl.kernel,
        mesh=mesh,
        out_type=jax.ShapeDtypeStruct((NSC, NP, d), jnp.float32),
        scratch_types=[
            pltpu.VMEM((GCHUNK, CHUNK), jnp.int32),
            pltpu.VMEM((GCHUNK, CHUNK), jnp.int32),
            pltpu.VMEM((NBUF, CHUNK, d), jnp.float32),
            pltpu.VMEM_SHARED((NP, d), jnp.float32),
            pltpu.SemaphoreType.DMA((NBUF,)),
            pltpu.SemaphoreType.DMA((NBUF,)),
        ],
    )
    def prop(t_hbm, src_hbm, dst_hbm, zero_hbm, out_hbm,
             src_v, dst_v, rows_v, acc_sh, gsem, ssem):
        c = lax.axis_index("c")
        s = lax.axis_index("s")
        wid = c * NTILE + s
        r0 = s * ROWS_PT

        pltpu.sync_copy(zero_hbm, rows_v.at[0])
        for z in range(ZCH):
            pltpu.sync_copy(rows_v.at[0], acc_sh.at[pl.ds(r0 + z * CHUNK, CHUNK)])
        plsc.subcore_barrier()

        @pl.loop(0, NGRP)
        def _(g):
            g8 = pl.multiple_of(g * GCHUNK, GCHUNK)
            pltpu.sync_copy(src_hbm.at[wid, pl.ds(g8, GCHUNK)], src_v)
            pltpu.sync_copy(dst_hbm.at[wid, pl.ds(g8, GCHUNK)], dst_v)
            for p in range(NBUF - 1):
                pltpu.async_copy(t_hbm.at[src_v.at[p]], rows_v.at[p],
                                 gsem.at[p])
            for jj in range(GCHUNK):
                b = jj % NBUF
                k = jj + NBUF - 1
                if k < GCHUNK:
                    kb = k % NBUF
                    if jj >= 1:
                        pltpu.make_async_copy(
                            rows_v.at[kb], acc_sh.at[dst_v.at[jj - 1]],
                            ssem.at[kb]).wait()
                    pltpu.async_copy(t_hbm.at[src_v.at[k]], rows_v.at[kb],
                                     gsem.at[kb])
                pltpu.make_async_copy(
                    t_hbm.at[src_v.at[jj]], rows_v.at[b], gsem.at[b]).wait()
                pltpu.async_copy(rows_v.at[b], acc_sh.at[dst_v.at[jj]],
                                 ssem.at[b], add=True)
            for e in range(GCHUNK - NBUF, GCHUNK):
                pltpu.make_async_copy(
                    rows_v.at[e % NBUF], acc_sh.at[dst_v.at[e]],
                    ssem.at[e % NBUF]).wait()

        plsc.subcore_barrier()

        for z in range(ZCH):
            rz = r0 + z * CHUNK
            pltpu.sync_copy(acc_sh.at[pl.ds(rz, CHUNK)], rows_v.at[0])
            pltpu.sync_copy(rows_v.at[0], out_hbm.at[c, pl.ds(rz, CHUNK)])

    return prop


def _pad_edges(idx, fill):
    w = idx.reshape(NW, EPW)
    pad = jnp.full((NW, EPW_P - EPW), fill, jnp.int32)
    return jnp.concatenate([w, pad], axis=1).reshape(NW, NCHUNK, CHUNK)


def kernel(x, edge_index, W1, b1, W2, b2, W3, b3):
    src = _pad_edges(edge_index[0].astype(jnp.int32), 0)
    dst = _pad_edges(edge_index[1].astype(jnp.int32), N)
    z128 = jnp.zeros((CHUNK, 128), jnp.float32)
    W3p = jnp.concatenate([W3, jnp.zeros((128, 64), jnp.float32)], axis=1)

    prop128 = _make_prop(128)

    t1 = _mm(x, W1)
    s1 = prop128(t1, src, dst, z128)
    t2 = _fused_mm(s1, b1, W2)
    s2 = prop128(t2, src, dst, z128)
    t3 = _fused_mm(s2, b2, W3p)
    s3 = prop128(t3, src, dst, z128)
    return _fin(s3[:, :, :64], b3)


# speedup vs baseline: 3.7440x; 1.0304x over previous
import functools

import jax
import jax.numpy as jnp
from jax import lax
from jax.experimental import pallas as pl
from jax.experimental.pallas import tpu as pltpu
from jax.experimental.pallas import tpu_sc as plsc

N = 10000
E = 320000
NSC = 2
NTILE = 16
NW = NSC * NTILE
CHUNK = 128
EPW = E // NW
EPW_P = 10240
NCHUNK = EPW_P // CHUNK
GCHUNK = 16
NGRP = NCHUNK // GCHUNK
NP = 10240
ROWS_PT = NP // NTILE
ZCH = ROWS_PT // CHUNK
NBUF = 2

TM = 1000


def _mm_body(x_ref, w_ref, o_ref):
    o_ref[...] = jnp.dot(x_ref[...], w_ref[...],
                         preferred_element_type=jnp.float32)


def _fused_mm_body(acc_ref, b_ref, w_ref, o_ref):
    s = acc_ref[0] + acc_ref[1] + b_ref[...]
    h = jnp.where(s >= 0, s, 0.25 * s)
    o_ref[...] = jnp.dot(h, w_ref[...], preferred_element_type=jnp.float32)


def _fin_body(acc_ref, b_ref, o_ref):
    s = acc_ref[0] + acc_ref[1] + b_ref[...]
    o_ref[...] = jnp.where(s >= 0, s, 0.25 * s)


def _mm(x, w):
    din, dout = w.shape
    return pl.pallas_call(
        _mm_body,
        grid=(N // TM,),
        in_specs=[
            pl.BlockSpec((TM, din), lambda i: (i, 0)),
            pl.BlockSpec((din, dout), lambda i: (0, 0)),
        ],
        out_specs=pl.BlockSpec((TM, dout), lambda i: (i, 0)),
        out_shape=jax.ShapeDtypeStruct((N, dout), jnp.float32),
    )(x, w)


def _fused_mm(acc, b, w):
    din, dout = w.shape
    return pl.pallas_call(
        _fused_mm_body,
        grid=(N // TM,),
        in_specs=[
            pl.BlockSpec((2, TM, din), lambda i: (0, i, 0)),
            pl.BlockSpec((1, din), lambda i: (0, 0)),
            pl.BlockSpec((din, dout), lambda i: (0, 0)),
        ],
        out_specs=pl.BlockSpec((TM, dout), lambda i: (i, 0)),
        out_shape=jax.ShapeDtypeStruct((N, dout), jnp.float32),
    )(acc[:, :N, :], b.reshape(1, din), w)


def _fin(acc, b):
    d = acc.shape[-1]
    return pl.pallas_call(
        _fin_body,
        grid=(N // TM,),
        in_specs=[
            pl.BlockSpec((2, TM, d), lambda i: (0, i, 0)),
            pl.BlockSpec((1, d), lambda i: (0, 0)),
        ],
        out_specs=pl.BlockSpec((TM, d), lambda i: (i, 0)),
        out_shape=jax.ShapeDtypeStruct((N, d), jnp.float32),
    )(acc[:, :N, :], b.reshape(1, d))


@functools.lru_cache(maxsize=None)
def _make_prop(d):
    mesh = plsc.VectorSubcoreMesh(core_axis_name="c", subcore_axis_name="s")

    @functools.partial(
        pl.kernel,
        mesh=mesh,
        out_type=jax.ShapeDtypeStruct((NSC, NP, d), jnp.float32),
        scratch_types=[
            pltpu.VMEM((2, GCHUNK, CHUNK), jnp.int32),
            pltpu.VMEM((NBUF, CHUNK, d), jnp.float32),
            pltpu.VMEM_SHARED((NP, d), jnp.float32),
            pltpu.SemaphoreType.DMA((NBUF,)),
            pltpu.SemaphoreType.DMA((NBUF,)),
        ],
    )
    def prop(t_hbm, idx_hbm, zero_hbm, out_hbm,
             ix_v, rows_v, acc_sh, gsem, ssem):
        c = lax.axis_index("c")
        s = lax.axis_index("s")
        wid = c * NTILE + s
        r0 = s * ROWS_PT

        pltpu.sync_copy(zero_hbm, rows_v.at[0])
        for z in range(ZCH):
            pltpu.sync_copy(rows_v.at[0], acc_sh.at[pl.ds(r0 + z * CHUNK, CHUNK)])
        plsc.subcore_barrier()

        @pl.loop(0, NGRP)
        def _(g):
            pltpu.sync_copy(idx_hbm.at[wid, g], ix_v)
            for p in range(NBUF - 1):
                pltpu.async_copy(t_hbm.at[ix_v.at[0, p]], rows_v.at[p],
                                 gsem.at[p])
            for jj in range(GCHUNK):
                b = jj % NBUF
                k = jj + NBUF - 1
                if k < GCHUNK:
                    kb = k % NBUF
                    if jj >= 1:
                        pltpu.make_async_copy(
                            rows_v.at[kb], acc_sh.at[ix_v.at[1, jj - 1]],
                            ssem.at[kb]).wait()
                    pltpu.async_copy(t_hbm.at[ix_v.at[0, k]], rows_v.at[kb],
                                     gsem.at[kb])
                pltpu.make_async_copy(
                    t_hbm.at[ix_v.at[0, jj]], rows_v.at[b], gsem.at[b]).wait()
                pltpu.async_copy(rows_v.at[b], acc_sh.at[ix_v.at[1, jj]],
                                 ssem.at[b], add=True)
            for e in range(GCHUNK - NBUF, GCHUNK):
                pltpu.make_async_copy(
                    rows_v.at[e % NBUF], acc_sh.at[ix_v.at[1, e]],
                    ssem.at[e % NBUF]).wait()

        plsc.subcore_barrier()

        for z in range(ZCH):
            rz = r0 + z * CHUNK
            pltpu.sync_copy(acc_sh.at[pl.ds(rz, CHUNK)], rows_v.at[0])
            pltpu.sync_copy(rows_v.at[0], out_hbm.at[c, pl.ds(rz, CHUNK)])

    return prop


def _pad_edges(idx, fill):
    w = idx.reshape(NW, EPW)
    pad = jnp.full((NW, EPW_P - EPW), fill, jnp.int32)
    return jnp.concatenate([w, pad], axis=1).reshape(NW, NCHUNK, CHUNK)


def kernel(x, edge_index, W1, b1, W2, b2, W3, b3):
    src = _pad_edges(edge_index[0].astype(jnp.int32), 0)
    dst = _pad_edges(edge_index[1].astype(jnp.int32), N)
    idx = jnp.stack(
        [src.reshape(NW, NGRP, GCHUNK, CHUNK),
         dst.reshape(NW, NGRP, GCHUNK, CHUNK)], axis=2)
    z128 = jnp.zeros((CHUNK, 128), jnp.float32)
    W3p = jnp.concatenate([W3, jnp.zeros((128, 64), jnp.float32)], axis=1)

    prop128 = _make_prop(128)

    t1 = _mm(x, W1)
    s1 = prop128(t1, idx, z128)
    t2 = _fused_mm(s1, b1, W2)
    s2 = prop128(t2, idx, z128)
    t3 = _fused_mm(s2, b2, W3p)
    s3 = prop128(t3, idx, z128)
    return _fin(s3[:, :, :64], b3)


# speedup vs baseline: 3.7629x; 1.0050x over previous
import functools

import jax
import jax.numpy as jnp
from jax import lax
from jax.experimental import pallas as pl
from jax.experimental.pallas import tpu as pltpu
from jax.experimental.pallas import tpu_sc as plsc

N = 10000
E = 320000
NSC = 2
NTILE = 16
NW = NSC * NTILE
CHUNK = 128
EPW = E // NW
EPW_P = 10240
NCHUNK = EPW_P // CHUNK
GCHUNK = 16
NGRP = NCHUNK // GCHUNK
NP = 10240
ROWS_PT = NP // NTILE
ZCH = ROWS_PT // CHUNK
NBUF = 2

TM = 1000


def _mm_body(x_ref, w_ref, o_ref):
    o_ref[...] = jnp.dot(x_ref[...], w_ref[...],
                         preferred_element_type=jnp.float32)


def _fused_mm_body(acc_ref, b_ref, w_ref, o_ref):
    s = acc_ref[0] + acc_ref[1] + b_ref[...]
    h = jnp.where(s >= 0, s, 0.25 * s)
    o_ref[...] = jnp.dot(h, w_ref[...], preferred_element_type=jnp.float32)


def _fin_body(acc_ref, b_ref, o_ref):
    s = acc_ref[0] + acc_ref[1] + b_ref[...]
    o_ref[...] = jnp.where(s >= 0, s, 0.25 * s)


def _mm(x, w):
    din, dout = w.shape
    return pl.pallas_call(
        _mm_body,
        grid=(N // TM,),
        in_specs=[
            pl.BlockSpec((TM, din), lambda i: (i, 0)),
            pl.BlockSpec((din, dout), lambda i: (0, 0)),
        ],
        out_specs=pl.BlockSpec((TM, dout), lambda i: (i, 0)),
        out_shape=jax.ShapeDtypeStruct((N, dout), jnp.float32),
    )(x, w)


def _fused_mm(acc, b, w):
    din, dout = w.shape
    return pl.pallas_call(
        _fused_mm_body,
        grid=(N // TM,),
        in_specs=[
            pl.BlockSpec((2, TM, din), lambda i: (0, i, 0)),
            pl.BlockSpec((1, din), lambda i: (0, 0)),
            pl.BlockSpec((din, dout), lambda i: (0, 0)),
        ],
        out_specs=pl.BlockSpec((TM, dout), lambda i: (i, 0)),
        out_shape=jax.ShapeDtypeStruct((N, dout), jnp.float32),
    )(acc[:, :N, :], b.reshape(1, din), w)


def _fin(acc, b):
    d = acc.shape[-1]
    return pl.pallas_call(
        _fin_body,
        grid=(N // TM,),
        in_specs=[
            pl.BlockSpec((2, TM, d), lambda i: (0, i, 0)),
            pl.BlockSpec((1, d), lambda i: (0, 0)),
        ],
        out_specs=pl.BlockSpec((TM, d), lambda i: (i, 0)),
        out_shape=jax.ShapeDtypeStruct((N, d), jnp.float32),
    )(acc[:, :N, :], b.reshape(1, d))


@functools.lru_cache(maxsize=None)
def _make_prop(d):
    mesh = plsc.VectorSubcoreMesh(core_axis_name="c", subcore_axis_name="s")

    @functools.partial(
        pl.kernel,
        mesh=mesh,
        out_type=jax.ShapeDtypeStruct((NSC, NP, d), jnp.float32),
        scratch_types=[
            pltpu.VMEM((2, 2, GCHUNK, CHUNK), jnp.int32),
            pltpu.VMEM((NBUF, CHUNK, d), jnp.float32),
            pltpu.VMEM_SHARED((NP, d), jnp.float32),
            pltpu.SemaphoreType.DMA((NBUF,)),
            pltpu.SemaphoreType.DMA((NBUF,)),
            pltpu.SemaphoreType.DMA((2,)),
        ],
    )
    def prop(t_hbm, idx_hbm, zero_hbm, out_hbm,
             ix_v, rows_v, acc_sh, gsem, ssem, isem):
        c = lax.axis_index("c")
        s = lax.axis_index("s")
        wid = c * NTILE + s
        r0 = s * ROWS_PT

        pltpu.async_copy(idx_hbm.at[wid, 0], ix_v.at[0], isem.at[0])
        pltpu.sync_copy(zero_hbm, acc_sh.at[pl.ds(r0, ROWS_PT)])
        plsc.subcore_barrier()

        @pl.loop(0, NGRP)
        def _(g):
            gb = g % 2
            ix = ix_v.at[gb]
            pltpu.make_async_copy(
                idx_hbm.at[wid, g], ix, isem.at[gb]).wait()

            @pl.when(g + 1 < NGRP)
            def _():
                pltpu.async_copy(idx_hbm.at[wid, g + 1], ix_v.at[1 - gb],
                                 isem.at[1 - gb])

            for p in range(NBUF - 1):
                pltpu.async_copy(t_hbm.at[ix.at[0, p]], rows_v.at[p],
                                 gsem.at[p])
            for jj in range(GCHUNK):
                b = jj % NBUF
                k = jj + NBUF - 1
                if k < GCHUNK:
                    kb = k % NBUF
                    if jj >= 1:
                        pltpu.make_async_copy(
                            rows_v.at[kb], acc_sh.at[ix.at[1, jj - 1]],
                            ssem.at[kb]).wait()
                    pltpu.async_copy(t_hbm.at[ix.at[0, k]], rows_v.at[kb],
                                     gsem.at[kb])
                pltpu.make_async_copy(
                    t_hbm.at[ix.at[0, jj]], rows_v.at[b], gsem.at[b]).wait()
                pltpu.async_copy(rows_v.at[b], acc_sh.at[ix.at[1, jj]],
                                 ssem.at[b], add=True)
            for e in range(GCHUNK - NBUF, GCHUNK):
                pltpu.make_async_copy(
                    rows_v.at[e % NBUF], acc_sh.at[ix.at[1, e]],
                    ssem.at[e % NBUF]).wait()

        plsc.subcore_barrier()

        pltpu.sync_copy(acc_sh.at[pl.ds(r0, ROWS_PT)],
                        out_hbm.at[c, pl.ds(r0, ROWS_PT)])

    return prop


def _pad_edges(idx, fill):
    w = idx.reshape(NW, EPW)
    pad = jnp.full((NW, EPW_P - EPW), fill, jnp.int32)
    return jnp.concatenate([w, pad], axis=1).reshape(NW, NCHUNK, CHUNK)


def kernel(x, edge_index, W1, b1, W2, b2, W3, b3):
    src = _pad_edges(edge_index[0].astype(jnp.int32), 0)
    dst = _pad_edges(edge_index[1].astype(jnp.int32), N)
    idx = jnp.stack(
        [src.reshape(NW, NGRP, GCHUNK, CHUNK),
         dst.reshape(NW, NGRP, GCHUNK, CHUNK)], axis=2)
    z128 = jnp.zeros((ROWS_PT, 128), jnp.float32)
    W3p = jnp.concatenate([W3, jnp.zeros((128, 64), jnp.float32)], axis=1)

    prop128 = _make_prop(128)

    t1 = _mm(x, W1)
    s1 = prop128(t1, idx, z128)
    t2 = _fused_mm(s1, b1, W2)
    s2 = prop128(t2, idx, z128)
    t3 = _fused_mm(s2, b2, W3p)
    s3 = prop128(t3, idx, z128)
    return _fin(s3[:, :, :64], b3)
